# Initial kernel scaffold; baseline (speedup 1.0000x reference)
import jax
import jax.numpy as jnp
from jax.experimental import pallas as pl


def kernel(affinity_pred, affinity_init):
    raise NotImplementedError("write your pallas kernel here")


# speedup vs baseline: 12.0355x; 12.0355x over previous
import functools

import jax
import jax.numpy as jnp
from jax.experimental import pallas as pl

_N = 4096
_BLOCK = 256
_K = 5


def _loss_block_kernel(pred_ref, init_ref, out_ref):
    i = pl.program_id(0)
    pred = pred_ref[...]
    init = init_ref[...]
    b, n = pred.shape
    col = jax.lax.broadcasted_iota(jnp.int32, (b, n), 1)
    row = jax.lax.broadcasted_iota(jnp.int32, (b, n), 0) + i * b
    diag = col == row
    expp = jnp.where(diag, 0.0, jnp.exp(pred))
    denom = jnp.sum(expp, axis=1, keepdims=True)
    neg = jnp.float32(-jnp.inf)
    vals = jnp.where(diag, neg, init)
    acc = jnp.zeros((b, 1), jnp.float32)
    for _ in range(_K):
        m = jnp.max(vals, axis=1, keepdims=True)
        eq = vals == m
        idx = jnp.min(jnp.where(eq, col, n), axis=1, keepdims=True)
        sel = col == idx
        pos_e = jnp.sum(jnp.where(sel, expp, 0.0), axis=1, keepdims=True)
        acc += -jnp.log(pos_e / denom + 1e-8)
        vals = jnp.where(sel, neg, vals)
    block_sum = jnp.sum(acc)

    @pl.when(i == 0)
    def _():
        out_ref[...] = jnp.zeros((1, 1), jnp.float32)

    out_ref[...] += jnp.full((1, 1), block_sum * (1.0 / _N), jnp.float32)


@functools.partial(jax.jit)
def kernel(affinity_pred, affinity_init):
    grid = _N // _BLOCK
    out = pl.pallas_call(
        _loss_block_kernel,
        grid=(grid,),
        in_specs=[
            pl.BlockSpec((_BLOCK, _N), lambda i: (i, 0)),
            pl.BlockSpec((_BLOCK, _N), lambda i: (i, 0)),
        ],
        out_specs=pl.BlockSpec((1, 1), lambda i: (0, 0)),
        out_shape=jax.ShapeDtypeStruct((1, 1), jnp.float32),
    )(affinity_pred, affinity_init)
    return out[0, 0]
